# Initial kernel scaffold; baseline (speedup 1.0000x reference)
import jax
import jax.numpy as jnp
from jax.experimental import pallas as pl


def kernel(x, edge_index, W1, b1, W2, b2, Wl, bl, W3, b3, W4, b4):
    raise NotImplementedError("write your pallas kernel here")


# speedup vs baseline: 9.7656x; 9.7656x over previous
import functools

import jax
import jax.numpy as jnp
from jax import lax
from jax.experimental import pallas as pl
from jax.experimental.pallas import tpu as pltpu
from jax.experimental.pallas import tpu_sc as plsc

_NC = 2
_NS = 16
_CHUNK = 128
_DEG_W = 8
_DEG_TRASH = 512
_ZROWS = 32

_HIGH = lax.Precision.HIGHEST


def _dot(a, b):
    return lax.dot_general(a, b, (((1,), (0,)), ((), ())),
                           precision=_HIGH, preferred_element_type=jnp.float32)


def _dinv_of(degp):
    return 1.0 / jnp.sqrt(degp[:, 0:1] + 1.0)


def _zero_accum(zbuf, accum, rps):
    s = lax.axis_index("s")
    for k in range(rps // _ZROWS):
        pltpu.sync_copy(zbuf, accum.at[pl.ds(s * rps + k * _ZROWS, _ZROWS)])
    rem = rps % _ZROWS
    if rem:
        pltpu.sync_copy(
            zbuf.at[pl.ds(0, rem)],
            accum.at[pl.ds(s * rps + (rps - rem), rem)])


def _write_out(accum, out_hbm, rps):
    c = lax.axis_index("c")
    s = lax.axis_index("s")
    pltpu.sync_copy(accum.at[pl.ds(s * rps, rps)],
                    out_hbm.at[c, pl.ds(s * rps, rps)])


def _sc_deg(half, per):
    hpad = half + _DEG_TRASH
    rps = half // _NS
    rows_l = per // _CHUNK

    def body(dst_hbm, ones_hbm, zeros_hbm, out_hbm, dst_l, sbuf_d,
             ones_v, zbuf, accum):
        c = lax.axis_index("c")
        s = lax.axis_index("s")
        lo = c * half
        it = lax.iota(jnp.int32, 16)
        pltpu.sync_copy(ones_hbm, ones_v)
        pltpu.sync_copy(zeros_hbm, zbuf)
        pltpu.sync_copy(dst_hbm.at[s], sbuf_d)

        def row(r, carry):
            for t in range(_CHUNK // 16):
                dv = sbuf_d[pl.ds(r * _CHUNK + t * 16, 16)]
                m = (dv >= lo) & (dv < lo + half)
                trash = half + ((it * 24 + r * 8 + t) & (_DEG_TRASH - 1))
                dst_l[r, pl.ds(t * 16, 16)] = jnp.where(m, dv - lo, trash)
            return carry

        lax.fori_loop(0, rows_l, row, 0)
        _zero_accum(zbuf, accum, rps)
        pltpu.sync_copy(zbuf, accum.at[pl.ds(half + s * _ZROWS, _ZROWS)])
        plsc.subcore_barrier()

        def step(j, carry):
            pltpu.sync_copy(ones_v, accum.at[dst_l.at[j]], add=True)
            return carry

        lax.fori_loop(0, rows_l, step, 0)
        plsc.subcore_barrier()
        _write_out(accum, out_hbm, rps)

    return pl.kernel(
        body,
        out_type=jax.ShapeDtypeStruct((_NC, half, _DEG_W), jnp.float32),
        mesh=plsc.VectorSubcoreMesh(core_axis_name="c", subcore_axis_name="s"),
        compiler_params=pltpu.CompilerParams(use_tc_tiling_on_sc=False),
        scratch_types=[
            pltpu.VMEM((rows_l, _CHUNK), jnp.int32),
            pltpu.VMEM((per,), jnp.int32),
            pltpu.VMEM((_CHUNK, _DEG_W), jnp.float32),
            pltpu.VMEM((_ZROWS, _DEG_W), jnp.float32),
            pltpu.VMEM_SHARED((hpad, _DEG_W), jnp.float32),
        ],
    )


def _sc_agg(half, per, d, ytall, n, zspread, tc_tiling=False):
    rps = half // _NS
    rows_l = per // _CHUNK
    piece = per // 2

    def body(y_hbm, src_hbm, dst_hbm, out_hbm, src_l, dst_l, sbuf_s, sbuf_d,
             rows0, zbuf, accum, sem0):
        c = lax.axis_index("c")
        s = lax.axis_index("s")
        lo = c * half
        it = lax.iota(jnp.int32, 16)
        for r in range(_ZROWS):
            for t in range(d // 16):
                zbuf[r, pl.ds(t * 16, 16)] = jnp.zeros((16,), jnp.float32)

        for p in range(2):
            pltpu.sync_copy(src_hbm.at[s, pl.ds(p * piece, piece)], sbuf_s)
            pltpu.sync_copy(dst_hbm.at[s, pl.ds(p * piece, piece)], sbuf_d)

            def row(r, carry, _p=p):
                for t in range(_CHUNK // 16):
                    sv = sbuf_s[pl.ds(r * _CHUNK + t * 16, 16)]
                    dv = sbuf_d[pl.ds(r * _CHUNK + t * 16, 16)]
                    m = (dv >= lo) & (dv < lo + half)
                    zrow = n + ((it * 151 + r * 8 + t) & (zspread - 1))
                    spread = (it * 24 + r * 8 + t) & 4095
                    lr = _p * (piece // _CHUNK) + r
                    src_l[lr, pl.ds(t * 16, 16)] = jnp.where(m, sv, zrow)
                    dst_l[lr, pl.ds(t * 16, 16)] = jnp.where(m, dv - lo, spread)
                return carry

            lax.fori_loop(0, piece // _CHUNK, row, 0)

        _zero_accum(zbuf, accum, rps)
        plsc.subcore_barrier()

        def step(j, carry):
            cp = pltpu.make_async_copy(y_hbm.at[src_l.at[j]], rows0, sem0)
            cp.start()
            cp.wait()
            pltpu.sync_copy(rows0, accum.at[dst_l.at[j]], add=True)
            return carry

        lax.fori_loop(0, rows_l, step, 0)
        plsc.subcore_barrier()
        _write_out(accum, out_hbm, rps)

    return pl.kernel(
        body,
        out_type=jax.ShapeDtypeStruct((_NC, half, d), jnp.float32),
        mesh=plsc.VectorSubcoreMesh(core_axis_name="c", subcore_axis_name="s"),
        compiler_params=pltpu.CompilerParams(use_tc_tiling_on_sc=tc_tiling),
        scratch_types=[
            pltpu.VMEM((rows_l, _CHUNK), jnp.int32),
            pltpu.VMEM((rows_l, _CHUNK), jnp.int32),
            pltpu.VMEM((piece,), jnp.int32),
            pltpu.VMEM((piece,), jnp.int32),
            pltpu.VMEM((_CHUNK, d), jnp.float32),
            pltpu.VMEM((_ZROWS, d), jnp.float32),
            pltpu.VMEM_SHARED((half, d), jnp.float32),
            pltpu.SemaphoreType.DMA,
        ],
    )


def _tc_pre_body(x_ref, nz_ref, degp_ref, w1_ref, y1_ref):
    dinv = _dinv_of(degp_ref[...])
    xa = x_ref[...] + nz_ref[...]
    y1_ref[...] = _dot(xa, w1_ref[...]) * dinv


def _tc_mid_body(n, blk, degp_ref, z1_ref, y1_ref, b1_ref, w2_ref,
                 x1_ref, y2_ref):
    dinv = _dinv_of(degp_ref[...])
    x1 = (z1_ref[...] + y1_ref[...]) * dinv + b1_ref[...]
    x1_ref[...] = x1
    i = pl.program_id(0)
    row = i * blk + lax.broadcasted_iota(jnp.int32, (blk, 1), 0)
    y2_ref[...] = jnp.where(row < n, _dot(x1, w2_ref[...]) * dinv, 0.0)


def _tc_tail_body(degp_ref, z2_ref, y2_ref, b2_ref, x1_ref, wl1_ref, wl2_ref,
                  bl_ref, w3_ref, b3_ref, w4_ref, b4_ref, hp_ref, ch_ref):
    dinv = _dinv_of(degp_ref[...])
    x2 = (z2_ref[...] + y2_ref[...]) * dinv + b2_ref[...]
    hp = _dot(x1_ref[...], wl1_ref[...]) + _dot(x2, wl2_ref[...]) + bl_ref[...]
    hp_ref[...] = hp
    t = jnp.maximum(_dot(hp, w3_ref[...]) + b3_ref[...], 0.0)
    ch_ref[...] = _dot(t, w4_ref[...]) + b4_ref[...]


def _full(shape):
    nd = len(shape)
    return pl.BlockSpec(shape, lambda i, _n=nd: (0,) * _n)


def _rows(blk, w):
    return pl.BlockSpec((blk, w), lambda i: (i, 0))


def kernel(x, edge_index, W1, b1, W2, b2, Wl, bl, W3, b3, W4, b4):
    n, d_in = x.shape
    e = edge_index.shape[1]
    d1 = W1.shape[1]
    d2 = W2.shape[1]
    dl = Wl.shape[1]
    d3 = W3.shape[1]
    d4 = W4.shape[1]

    half = -(-n // 64) * 32
    nsc = 2 * half
    npad = -(-max(nsc, n) // 1024) * 1024
    zspread = 2048
    ytall = npad + zspread
    blk = npad // 16

    k = jax.random.key(42)
    n1 = jax.random.normal(jax.random.fold_in(k, 1), x.shape, x.dtype) * 0.1
    n2 = jax.random.normal(jax.random.fold_in(k, 2), x.shape, x.dtype) * 0.1
    nz = n1 + n2

    xpad = jnp.pad(x, ((0, npad - n), (0, 0)))
    nzpad = jnp.pad(nz, ((0, npad - n), (0, 0)))

    per = -(-e // (_NS * _CHUNK * 2)) * _CHUNK * 2
    epad = _NS * per
    pad_rows = (n + jnp.arange(epad - e, dtype=jnp.int32) % (ytall - n - 256))
    src2 = jnp.concatenate([edge_index[0], pad_rows]).reshape(_NS, per)
    dst2 = jnp.concatenate([edge_index[1], pad_rows]).reshape(_NS, per)

    ones8 = jnp.ones((_CHUNK, _DEG_W), jnp.float32)
    zeros8 = jnp.zeros((_ZROWS, _DEG_W), jnp.float32)

    b1r, b2r, blr, b3r, b4r = (v.reshape(1, -1) for v in (b1, b2, bl, b3, b4))

    degp = _sc_deg(half, per)(dst2, ones8, zeros8)
    deg = jnp.pad(degp.reshape(nsc, _DEG_W), ((0, npad - nsc), (0, 0)))

    grid = (npad // blk,)
    y1 = pl.pallas_call(
        _tc_pre_body,
        grid=grid,
        in_specs=[_rows(blk, d_in), _rows(blk, d_in), _rows(blk, _DEG_W),
                  _full((d_in, d1))],
        out_specs=_rows(blk, d1),
        out_shape=jax.ShapeDtypeStruct((npad, d1), jnp.float32),
    )(xpad, nzpad, deg, W1)

    y1t = jnp.pad(y1, ((0, ytall - npad), (0, 0)))
    z1p = _sc_agg(half, per, d1, ytall, n, zspread)(y1t, src2, dst2)
    z1 = jnp.pad(z1p.reshape(nsc, d1), ((0, npad - nsc), (0, 0)))

    x1, y2 = pl.pallas_call(
        functools.partial(_tc_mid_body, n, blk),
        grid=grid,
        in_specs=[_rows(blk, _DEG_W), _rows(blk, d1), _rows(blk, d1),
                  _full((1, d1)), _full((d1, d2))],
        out_specs=[_rows(blk, d1), _rows(blk, d2)],
        out_shape=[jax.ShapeDtypeStruct((npad, d1), jnp.float32),
                   jax.ShapeDtypeStruct((npad, d2), jnp.float32)],
    )(deg, z1, y1, b1r, W2)

    y2t = jnp.pad(y2, ((0, ytall - npad), (0, 0)))
    dh = d2 // 2
    agg2 = _sc_agg(half, per, dh, ytall, n, zspread)
    z2pa = agg2(y2t[:, :dh], src2, dst2)
    z2pb = agg2(y2t[:, dh:], src2, dst2)
    z2 = jnp.concatenate(
        [jnp.pad(z2pa.reshape(nsc, dh), ((0, npad - nsc), (0, 0))),
         jnp.pad(z2pb.reshape(nsc, dh), ((0, npad - nsc), (0, 0)))], axis=1)

    hp, chm = pl.pallas_call(
        _tc_tail_body,
        grid=grid,
        in_specs=[_rows(blk, _DEG_W), _rows(blk, d2), _rows(blk, d2),
                  _full((1, d2)), _rows(blk, d1), _full((d1, dl)),
                  _full((d2, dl)), _full((1, dl)), _full((dl, d3)),
                  _full((1, d3)), _full((d3, d4)), _full((1, d4))],
        out_specs=[_rows(blk, dl), _rows(blk, d4)],
        out_shape=[jax.ShapeDtypeStruct((npad, dl), jnp.float32),
                   jax.ShapeDtypeStruct((npad, d4), jnp.float32)],
    )(deg, z2, y2, b2r, x1, Wl[:d1], Wl[d1:], blr, W3, b3r, W4, b4r)

    h = hp[:n][None]
    c = chm[:n][None]
    return (h, h, c, c)


# speedup vs baseline: 13.9627x; 1.4298x over previous
import functools

import jax
import jax.numpy as jnp
from jax import lax
from jax.experimental import pallas as pl
from jax.experimental.pallas import tpu as pltpu
from jax.experimental.pallas import tpu_sc as plsc

_NC = 2
_NS = 16
_CHUNK = 128
_DEG_W = 8
_DEG_TRASH = 512
_ZROWS = 32

_HIGH = lax.Precision.HIGHEST


def _dot(a, b):
    return lax.dot_general(a, b, (((1,), (0,)), ((), ())),
                           precision=_HIGH, preferred_element_type=jnp.float32)


def _dinv_of(degp):
    return 1.0 / jnp.sqrt(degp[:, 0:1] + 1.0)


def _zero_accum(zbuf, accum, rps):
    s = lax.axis_index("s")
    for k in range(rps // _ZROWS):
        pltpu.sync_copy(zbuf, accum.at[pl.ds(s * rps + k * _ZROWS, _ZROWS)])
    rem = rps % _ZROWS
    if rem:
        pltpu.sync_copy(
            zbuf.at[pl.ds(0, rem)],
            accum.at[pl.ds(s * rps + (rps - rem), rem)])


def _write_out(accum, out_hbm, rps):
    c = lax.axis_index("c")
    s = lax.axis_index("s")
    pltpu.sync_copy(accum.at[pl.ds(s * rps, rps)],
                    out_hbm.at[c, pl.ds(s * rps, rps)])


def _sc_deg(half, per):
    hpad = half + _DEG_TRASH
    rps = half // _NS
    rows_l = per // _CHUNK

    def body(dst_hbm, ones_hbm, zeros_hbm, out_hbm, dst_l, sbuf_d,
             ones_v, zbuf, accum):
        c = lax.axis_index("c")
        s = lax.axis_index("s")
        lo = c * half
        it = lax.iota(jnp.int32, 16)
        pltpu.sync_copy(ones_hbm, ones_v)
        pltpu.sync_copy(zeros_hbm, zbuf)
        pltpu.sync_copy(dst_hbm.at[s], sbuf_d)

        def row(r, carry):
            for t in range(_CHUNK // 16):
                dv = sbuf_d[pl.ds(r * _CHUNK + t * 16, 16)]
                m = (dv >= lo) & (dv < lo + half)
                trash = half + ((it * 24 + r * 8 + t) & (_DEG_TRASH - 1))
                dst_l[r, pl.ds(t * 16, 16)] = jnp.where(m, dv - lo, trash)
            return carry

        lax.fori_loop(0, rows_l, row, 0)
        _zero_accum(zbuf, accum, rps)
        pltpu.sync_copy(zbuf, accum.at[pl.ds(half + s * _ZROWS, _ZROWS)])
        plsc.subcore_barrier()

        def step(j, carry):
            pltpu.sync_copy(ones_v, accum.at[dst_l.at[j]], add=True)
            return carry

        lax.fori_loop(0, rows_l, step, 0)
        plsc.subcore_barrier()
        _write_out(accum, out_hbm, rps)

    return pl.kernel(
        body,
        out_type=jax.ShapeDtypeStruct((_NC, half, _DEG_W), jnp.float32),
        mesh=plsc.VectorSubcoreMesh(core_axis_name="c", subcore_axis_name="s"),
        compiler_params=pltpu.CompilerParams(use_tc_tiling_on_sc=False),
        scratch_types=[
            pltpu.VMEM((rows_l, _CHUNK), jnp.int32),
            pltpu.VMEM((per,), jnp.int32),
            pltpu.VMEM((_CHUNK, _DEG_W), jnp.float32),
            pltpu.VMEM((_ZROWS, _DEG_W), jnp.float32),
            pltpu.VMEM_SHARED((hpad, _DEG_W), jnp.float32),
        ],
    )


def _sc_agg(half, per, d, ytall, n, zspread, tc_tiling=False):
    rps = half // _NS
    rows_l = per // _CHUNK
    piece = per // 2

    def body(y_hbm, src_hbm, dst_hbm, out_hbm, src_l, dst_l, sbuf_s, sbuf_d,
             rows0, rows1, zbuf, accum, sem0, sem1):
        c = lax.axis_index("c")
        s = lax.axis_index("s")
        lo = c * half
        it = lax.iota(jnp.int32, 16)
        for r in range(_ZROWS):
            for t in range(d // 16):
                zbuf[r, pl.ds(t * 16, 16)] = jnp.zeros((16,), jnp.float32)

        for p in range(2):
            pltpu.sync_copy(src_hbm.at[s, pl.ds(p * piece, piece)], sbuf_s)
            pltpu.sync_copy(dst_hbm.at[s, pl.ds(p * piece, piece)], sbuf_d)

            def row(r, carry, _p=p):
                for t in range(_CHUNK // 16):
                    sv = sbuf_s[pl.ds(r * _CHUNK + t * 16, 16)]
                    dv = sbuf_d[pl.ds(r * _CHUNK + t * 16, 16)]
                    m = (dv >= lo) & (dv < lo + half)
                    zrow = n + ((it * 151 + r * 8 + t) & (zspread - 1))
                    spread = (it * 24 + r * 8 + t) & 4095
                    lr = _p * (piece // _CHUNK) + r
                    src_l[lr, pl.ds(t * 16, 16)] = jnp.where(m, sv, zrow)
                    dst_l[lr, pl.ds(t * 16, 16)] = jnp.where(m, dv - lo, spread)
                return carry

            lax.fori_loop(0, piece // _CHUNK, row, 0)

        _zero_accum(zbuf, accum, rps)
        plsc.subcore_barrier()

        pltpu.make_async_copy(y_hbm.at[src_l.at[0]], rows0, sem0).start()

        def pair(u, carry):
            j0 = 2 * u
            pltpu.make_async_copy(
                y_hbm.at[src_l.at[j0 + 1]], rows1, sem1).start()
            pltpu.make_async_copy(
                y_hbm.at[src_l.at[j0]], rows0, sem0).wait()
            pltpu.sync_copy(rows0, accum.at[dst_l.at[j0]], add=True)

            @pl.when(u + 1 < rows_l // 2)
            def _():
                pltpu.make_async_copy(
                    y_hbm.at[src_l.at[j0 + 2]], rows0, sem0).start()

            pltpu.make_async_copy(
                y_hbm.at[src_l.at[j0 + 1]], rows1, sem1).wait()
            pltpu.sync_copy(rows1, accum.at[dst_l.at[j0 + 1]], add=True)
            return carry

        lax.fori_loop(0, rows_l // 2, pair, 0)
        plsc.subcore_barrier()
        _write_out(accum, out_hbm, rps)

    return pl.kernel(
        body,
        out_type=jax.ShapeDtypeStruct((_NC, half, d), jnp.float32),
        mesh=plsc.VectorSubcoreMesh(core_axis_name="c", subcore_axis_name="s"),
        compiler_params=pltpu.CompilerParams(use_tc_tiling_on_sc=tc_tiling),
        scratch_types=[
            pltpu.VMEM((rows_l, _CHUNK), jnp.int32),
            pltpu.VMEM((rows_l, _CHUNK), jnp.int32),
            pltpu.VMEM((piece,), jnp.int32),
            pltpu.VMEM((piece,), jnp.int32),
            pltpu.VMEM((_CHUNK, d), jnp.float32),
            pltpu.VMEM((_CHUNK, d), jnp.float32),
            pltpu.VMEM((_ZROWS, d), jnp.float32),
            pltpu.VMEM_SHARED((half, d), jnp.float32),
            pltpu.SemaphoreType.DMA,
            pltpu.SemaphoreType.DMA,
        ],
    )


def _tc_pre_body(x_ref, nz_ref, degp_ref, w1_ref, y1_ref):
    dinv = _dinv_of(degp_ref[...])
    xa = x_ref[...] + nz_ref[...]
    y1_ref[...] = _dot(xa, w1_ref[...]) * dinv


def _tc_mid_body(n, blk, degp_ref, z1_ref, y1_ref, b1_ref, w2_ref,
                 x1_ref, y2_ref):
    dinv = _dinv_of(degp_ref[...])
    x1 = (z1_ref[...] + y1_ref[...]) * dinv + b1_ref[...]
    x1_ref[...] = x1
    i = pl.program_id(0)
    row = i * blk + lax.broadcasted_iota(jnp.int32, (blk, 1), 0)
    y2_ref[...] = jnp.where(row < n, _dot(x1, w2_ref[...]) * dinv, 0.0)


def _tc_tail_body(degp_ref, z2_ref, y2_ref, b2_ref, x1_ref, wl1_ref, wl2_ref,
                  bl_ref, w3_ref, b3_ref, w4_ref, b4_ref, hp_ref, ch_ref):
    dinv = _dinv_of(degp_ref[...])
    x2 = (z2_ref[...] + y2_ref[...]) * dinv + b2_ref[...]
    hp = _dot(x1_ref[...], wl1_ref[...]) + _dot(x2, wl2_ref[...]) + bl_ref[...]
    hp_ref[...] = hp
    t = jnp.maximum(_dot(hp, w3_ref[...]) + b3_ref[...], 0.0)
    ch_ref[...] = _dot(t, w4_ref[...]) + b4_ref[...]


def _full(shape):
    nd = len(shape)
    return pl.BlockSpec(shape, lambda i, _n=nd: (0,) * _n)


def _rows(blk, w):
    return pl.BlockSpec((blk, w), lambda i: (i, 0))


def kernel(x, edge_index, W1, b1, W2, b2, Wl, bl, W3, b3, W4, b4):
    n, d_in = x.shape
    e = edge_index.shape[1]
    d1 = W1.shape[1]
    d2 = W2.shape[1]
    dl = Wl.shape[1]
    d3 = W3.shape[1]
    d4 = W4.shape[1]

    half = -(-n // 64) * 32
    nsc = 2 * half
    npad = -(-max(nsc, n) // 1024) * 1024
    zspread = 2048
    ytall = npad + zspread
    blk = npad // 16

    k = jax.random.key(42)
    n1 = jax.random.normal(jax.random.fold_in(k, 1), x.shape, x.dtype) * 0.1
    n2 = jax.random.normal(jax.random.fold_in(k, 2), x.shape, x.dtype) * 0.1
    nz = n1 + n2

    xpad = jnp.pad(x, ((0, npad - n), (0, 0)))
    nzpad = jnp.pad(nz, ((0, npad - n), (0, 0)))

    per = -(-e // (_NS * _CHUNK * 2)) * _CHUNK * 2
    epad = _NS * per
    pad_rows = (n + jnp.arange(epad - e, dtype=jnp.int32) % (ytall - n - 256))
    src2 = jnp.concatenate([edge_index[0], pad_rows]).reshape(_NS, per)
    dst2 = jnp.concatenate([edge_index[1], pad_rows]).reshape(_NS, per)

    ones8 = jnp.ones((_CHUNK, _DEG_W), jnp.float32)
    zeros8 = jnp.zeros((_ZROWS, _DEG_W), jnp.float32)

    b1r, b2r, blr, b3r, b4r = (v.reshape(1, -1) for v in (b1, b2, bl, b3, b4))

    degp = _sc_deg(half, per)(dst2, ones8, zeros8)
    deg = jnp.pad(degp.reshape(nsc, _DEG_W), ((0, npad - nsc), (0, 0)))

    grid = (npad // blk,)
    y1 = pl.pallas_call(
        _tc_pre_body,
        grid=grid,
        in_specs=[_rows(blk, d_in), _rows(blk, d_in), _rows(blk, _DEG_W),
                  _full((d_in, d1))],
        out_specs=_rows(blk, d1),
        out_shape=jax.ShapeDtypeStruct((npad, d1), jnp.float32),
    )(xpad, nzpad, deg, W1)

    y1t = jnp.pad(y1, ((0, ytall - npad), (0, 0)))
    z1p = _sc_agg(half, per, d1, ytall, n, zspread)(y1t, src2, dst2)
    z1 = jnp.pad(z1p.reshape(nsc, d1), ((0, npad - nsc), (0, 0)))

    x1, y2 = pl.pallas_call(
        functools.partial(_tc_mid_body, n, blk),
        grid=grid,
        in_specs=[_rows(blk, _DEG_W), _rows(blk, d1), _rows(blk, d1),
                  _full((1, d1)), _full((d1, d2))],
        out_specs=[_rows(blk, d1), _rows(blk, d2)],
        out_shape=[jax.ShapeDtypeStruct((npad, d1), jnp.float32),
                   jax.ShapeDtypeStruct((npad, d2), jnp.float32)],
    )(deg, z1, y1, b1r, W2)

    y2t = jnp.pad(y2, ((0, ytall - npad), (0, 0)))
    dh = d2 // 2
    agg2 = _sc_agg(half, per, dh, ytall, n, zspread)
    z2pa = agg2(y2t[:, :dh], src2, dst2)
    z2pb = agg2(y2t[:, dh:], src2, dst2)
    z2 = jnp.concatenate(
        [jnp.pad(z2pa.reshape(nsc, dh), ((0, npad - nsc), (0, 0))),
         jnp.pad(z2pb.reshape(nsc, dh), ((0, npad - nsc), (0, 0)))], axis=1)

    hp, chm = pl.pallas_call(
        _tc_tail_body,
        grid=grid,
        in_specs=[_rows(blk, _DEG_W), _rows(blk, d2), _rows(blk, d2),
                  _full((1, d2)), _rows(blk, d1), _full((d1, dl)),
                  _full((d2, dl)), _full((1, dl)), _full((dl, d3)),
                  _full((1, d3)), _full((d3, d4)), _full((1, d4))],
        out_specs=[_rows(blk, dl), _rows(blk, d4)],
        out_shape=[jax.ShapeDtypeStruct((npad, dl), jnp.float32),
                   jax.ShapeDtypeStruct((npad, d4), jnp.float32)],
    )(deg, z2, y2, b2r, x1, Wl[:d1], Wl[d1:], blr, W3, b3r, W4, b4r)

    h = hp[:n][None]
    c = chm[:n][None]
    return (h, h, c, c)


# speedup vs baseline: 13.9906x; 1.0020x over previous
import functools

import jax
import jax.numpy as jnp
from jax import lax
from jax.experimental import pallas as pl
from jax.experimental.pallas import tpu as pltpu
from jax.experimental.pallas import tpu_sc as plsc

_NC = 2
_NS = 16
_CHUNK = 128
_DEG_W = 8
_DEG_TRASH = 512
_ZROWS = 32

_HIGH = lax.Precision.HIGHEST


def _dot(a, b):
    return lax.dot_general(a, b, (((1,), (0,)), ((), ())),
                           precision=_HIGH, preferred_element_type=jnp.float32)


def _dinv_of(degp):
    return 1.0 / jnp.sqrt(degp[:, 0:1] + 1.0)


def _zero_accum(zbuf, accum, rps):
    s = lax.axis_index("s")
    for k in range(rps // _ZROWS):
        pltpu.sync_copy(zbuf, accum.at[pl.ds(s * rps + k * _ZROWS, _ZROWS)])
    rem = rps % _ZROWS
    if rem:
        pltpu.sync_copy(
            zbuf.at[pl.ds(0, rem)],
            accum.at[pl.ds(s * rps + (rps - rem), rem)])


def _write_out(accum, out_hbm, rps):
    c = lax.axis_index("c")
    s = lax.axis_index("s")
    pltpu.sync_copy(accum.at[pl.ds(s * rps, rps)],
                    out_hbm.at[c, pl.ds(s * rps, rps)])


def _sc_deg(half, per):
    hpad = half + _DEG_TRASH
    rps = half // _NS
    rows_l = per // _CHUNK

    def body(dst_hbm, ones_hbm, zeros_hbm, out_hbm, dst_l, sbuf_d,
             ones_v, zbuf, accum):
        c = lax.axis_index("c")
        s = lax.axis_index("s")
        lo = c * half
        it = lax.iota(jnp.int32, 16)
        pltpu.sync_copy(ones_hbm, ones_v)
        pltpu.sync_copy(zeros_hbm, zbuf)
        pltpu.sync_copy(dst_hbm.at[s], sbuf_d)

        def row(r, carry):
            for t in range(_CHUNK // 16):
                dv = sbuf_d[pl.ds(r * _CHUNK + t * 16, 16)]
                m = (dv >= lo) & (dv < lo + half)
                trash = half + ((it * 24 + r * 8 + t) & (_DEG_TRASH - 1))
                dst_l[r, pl.ds(t * 16, 16)] = jnp.where(m, dv - lo, trash)
            return carry

        lax.fori_loop(0, rows_l, row, 0)
        _zero_accum(zbuf, accum, rps)
        pltpu.sync_copy(zbuf, accum.at[pl.ds(half + s * _ZROWS, _ZROWS)])
        plsc.subcore_barrier()

        def step(j, carry):
            pltpu.sync_copy(ones_v, accum.at[dst_l.at[j]], add=True)
            return carry

        lax.fori_loop(0, rows_l, step, 0)
        plsc.subcore_barrier()
        _write_out(accum, out_hbm, rps)

    return pl.kernel(
        body,
        out_type=jax.ShapeDtypeStruct((_NC, half, _DEG_W), jnp.float32),
        mesh=plsc.VectorSubcoreMesh(core_axis_name="c", subcore_axis_name="s"),
        compiler_params=pltpu.CompilerParams(use_tc_tiling_on_sc=False),
        scratch_types=[
            pltpu.VMEM((rows_l, _CHUNK), jnp.int32),
            pltpu.VMEM((per,), jnp.int32),
            pltpu.VMEM((_CHUNK, _DEG_W), jnp.float32),
            pltpu.VMEM((_ZROWS, _DEG_W), jnp.float32),
            pltpu.VMEM_SHARED((hpad, _DEG_W), jnp.float32),
        ],
    )


def _sc_agg(half, per, d, ytall, n, zspread, tc_tiling=False):
    rps = half // _NS
    rows_l = per // _CHUNK
    piece = per // 2

    def body(y_hbm, src_hbm, dst_hbm, out_hbm, src_l, dst_l, sbuf_s, sbuf_d,
             rows0, rows1, zbuf, accum, sem0, sem1):
        c = lax.axis_index("c")
        s = lax.axis_index("s")
        lo = c * half
        it = lax.iota(jnp.int32, 16)
        for r in range(_ZROWS):
            for t in range(d // 16):
                zbuf[r, pl.ds(t * 16, 16)] = jnp.zeros((16,), jnp.float32)

        for p in range(2):
            pltpu.sync_copy(src_hbm.at[s, pl.ds(p * piece, piece)], sbuf_s)
            pltpu.sync_copy(dst_hbm.at[s, pl.ds(p * piece, piece)], sbuf_d)

            def row(r, carry, _p=p):
                for t in range(_CHUNK // 16):
                    sv = sbuf_s[pl.ds(r * _CHUNK + t * 16, 16)]
                    dv = sbuf_d[pl.ds(r * _CHUNK + t * 16, 16)]
                    m = (dv >= lo) & (dv < lo + half)
                    zrow = n + ((it * 151 + r * 8 + t) & (zspread - 1))
                    spread = (it * 24 + r * 8 + t) & 4095
                    lr = _p * (piece // _CHUNK) + r
                    src_l[lr, pl.ds(t * 16, 16)] = jnp.where(m, sv, zrow)
                    dst_l[lr, pl.ds(t * 16, 16)] = jnp.where(m, dv - lo, spread)
                return carry

            lax.fori_loop(0, piece // _CHUNK, row, 0)

        _zero_accum(zbuf, accum, rps)
        plsc.subcore_barrier()

        pltpu.make_async_copy(y_hbm.at[src_l.at[0]], rows0, sem0).start()

        def pair(u, carry):
            j0 = 2 * u
            pltpu.make_async_copy(
                y_hbm.at[src_l.at[j0 + 1]], rows1, sem1).start()
            pltpu.make_async_copy(
                y_hbm.at[src_l.at[j0]], rows0, sem0).wait()
            pltpu.sync_copy(rows0, accum.at[dst_l.at[j0]], add=True)

            @pl.when(u + 1 < rows_l // 2)
            def _():
                pltpu.make_async_copy(
                    y_hbm.at[src_l.at[j0 + 2]], rows0, sem0).start()

            pltpu.make_async_copy(
                y_hbm.at[src_l.at[j0 + 1]], rows1, sem1).wait()
            pltpu.sync_copy(rows1, accum.at[dst_l.at[j0 + 1]], add=True)
            return carry

        lax.fori_loop(0, rows_l // 2, pair, 0)
        plsc.subcore_barrier()
        _write_out(accum, out_hbm, rps)

    return pl.kernel(
        body,
        out_type=jax.ShapeDtypeStruct((_NC, half, d), jnp.float32),
        mesh=plsc.VectorSubcoreMesh(core_axis_name="c", subcore_axis_name="s"),
        compiler_params=pltpu.CompilerParams(use_tc_tiling_on_sc=tc_tiling),
        scratch_types=[
            pltpu.VMEM((rows_l, _CHUNK), jnp.int32),
            pltpu.VMEM((rows_l, _CHUNK), jnp.int32),
            pltpu.VMEM((piece,), jnp.int32),
            pltpu.VMEM((piece,), jnp.int32),
            pltpu.VMEM((_CHUNK, d), jnp.float32),
            pltpu.VMEM((_CHUNK, d), jnp.float32),
            pltpu.VMEM((_ZROWS, d), jnp.float32),
            pltpu.VMEM_SHARED((half, d), jnp.float32),
            pltpu.SemaphoreType.DMA,
            pltpu.SemaphoreType.DMA,
        ],
    )


def _tc_pre_body(x_ref, nz_ref, degp_ref, w1_ref, y1_ref):
    dinv = _dinv_of(degp_ref[...])
    xa = x_ref[...] + nz_ref[...]
    y1_ref[...] = _dot(xa, w1_ref[...]) * dinv


def _tc_mid_body(n, blk, degp_ref, z1_ref, y1_ref, b1_ref, w2_ref,
                 x1_ref, y2_ref):
    dinv = _dinv_of(degp_ref[...])
    x1 = (z1_ref[...] + y1_ref[...]) * dinv + b1_ref[...]
    x1_ref[...] = x1
    i = pl.program_id(0)
    row = i * blk + lax.broadcasted_iota(jnp.int32, (blk, 1), 0)
    y2_ref[...] = jnp.where(row < n, _dot(x1, w2_ref[...]) * dinv, 0.0)


def _tc_tail_body(degp_ref, z2_ref, y2_ref, b2_ref, x1_ref, wl1_ref, wl2_ref,
                  bl_ref, w3_ref, b3_ref, w4_ref, b4_ref, hp_ref, ch_ref):
    dinv = _dinv_of(degp_ref[...])
    x2 = (z2_ref[...] + y2_ref[...]) * dinv + b2_ref[...]
    hp = _dot(x1_ref[...], wl1_ref[...]) + _dot(x2, wl2_ref[...]) + bl_ref[...]
    hp_ref[...] = hp
    t = jnp.maximum(_dot(hp, w3_ref[...]) + b3_ref[...], 0.0)
    ch_ref[...] = _dot(t, w4_ref[...]) + b4_ref[...]


def _full(shape):
    nd = len(shape)
    return pl.BlockSpec(shape, lambda i, _n=nd: (0,) * _n)


def _rows(blk, w):
    return pl.BlockSpec((blk, w), lambda i: (i, 0))


def kernel(x, edge_index, W1, b1, W2, b2, Wl, bl, W3, b3, W4, b4):
    n, d_in = x.shape
    e = edge_index.shape[1]
    d1 = W1.shape[1]
    d2 = W2.shape[1]
    dl = Wl.shape[1]
    d3 = W3.shape[1]
    d4 = W4.shape[1]

    npad = -(-n // 1024) * 1024
    half = npad // 2
    nsc = npad
    zspread = 2048
    ytall = npad + zspread
    blk = npad // 16

    k = jax.random.key(42)
    n1 = jax.random.normal(jax.random.fold_in(k, 1), x.shape, x.dtype) * 0.1
    n2 = jax.random.normal(jax.random.fold_in(k, 2), x.shape, x.dtype) * 0.1
    nz = n1 + n2

    xpad = jnp.pad(x, ((0, npad - n), (0, 0)))
    nzpad = jnp.pad(nz, ((0, npad - n), (0, 0)))

    per = -(-e // (_NS * _CHUNK * 2)) * _CHUNK * 2
    epad = _NS * per
    pad_rows = (n + jnp.arange(epad - e, dtype=jnp.int32) % (ytall - n - 256))
    src2 = jnp.concatenate([edge_index[0], pad_rows]).reshape(_NS, per)
    dst2 = jnp.concatenate([edge_index[1], pad_rows]).reshape(_NS, per)

    ones8 = jnp.ones((_CHUNK, _DEG_W), jnp.float32)
    zeros8 = jnp.zeros((_ZROWS, _DEG_W), jnp.float32)

    b1r, b2r, blr, b3r, b4r = (v.reshape(1, -1) for v in (b1, b2, bl, b3, b4))

    degp = _sc_deg(half, per)(dst2, ones8, zeros8)
    deg = jnp.pad(degp.reshape(nsc, _DEG_W), ((0, npad - nsc), (0, 0)))

    grid = (npad // blk,)
    y1 = pl.pallas_call(
        _tc_pre_body,
        grid=grid,
        in_specs=[_rows(blk, d_in), _rows(blk, d_in), _rows(blk, _DEG_W),
                  _full((d_in, d1))],
        out_specs=_rows(blk, d1),
        out_shape=jax.ShapeDtypeStruct((npad, d1), jnp.float32),
    )(xpad, nzpad, deg, W1)

    y1t = jnp.pad(y1, ((0, ytall - npad), (0, 0)))
    z1p = _sc_agg(half, per, d1, ytall, n, zspread)(y1t, src2, dst2)
    z1 = jnp.pad(z1p.reshape(nsc, d1), ((0, npad - nsc), (0, 0)))

    x1, y2 = pl.pallas_call(
        functools.partial(_tc_mid_body, n, blk),
        grid=grid,
        in_specs=[_rows(blk, _DEG_W), _rows(blk, d1), _rows(blk, d1),
                  _full((1, d1)), _full((d1, d2))],
        out_specs=[_rows(blk, d1), _rows(blk, d2)],
        out_shape=[jax.ShapeDtypeStruct((npad, d1), jnp.float32),
                   jax.ShapeDtypeStruct((npad, d2), jnp.float32)],
    )(deg, z1, y1, b1r, W2)

    y2t = jnp.pad(y2, ((0, ytall - npad), (0, 0)))
    dh = d2 // 2
    agg2 = _sc_agg(half, per, dh, ytall, n, zspread)
    z2pa = agg2(y2t[:, :dh], src2, dst2)
    z2pb = agg2(y2t[:, dh:], src2, dst2)
    z2 = jnp.concatenate(
        [jnp.pad(z2pa.reshape(nsc, dh), ((0, npad - nsc), (0, 0))),
         jnp.pad(z2pb.reshape(nsc, dh), ((0, npad - nsc), (0, 0)))], axis=1)

    hp, chm = pl.pallas_call(
        _tc_tail_body,
        grid=grid,
        in_specs=[_rows(blk, _DEG_W), _rows(blk, d2), _rows(blk, d2),
                  _full((1, d2)), _rows(blk, d1), _full((d1, dl)),
                  _full((d2, dl)), _full((1, dl)), _full((dl, d3)),
                  _full((1, d3)), _full((d3, d4)), _full((1, d4))],
        out_specs=[_rows(blk, dl), _rows(blk, d4)],
        out_shape=[jax.ShapeDtypeStruct((npad, dl), jnp.float32),
                   jax.ShapeDtypeStruct((npad, d4), jnp.float32)],
    )(deg, z2, y2, b2r, x1, Wl[:d1], Wl[d1:], blr, W3, b3r, W4, b4r)

    h = hp[:n][None]
    c = chm[:n][None]
    return (h, h, c, c)
